# baseline (speedup 1.0000x reference)
import functools

import jax
import jax.numpy as jnp
from jax import lax
from jax.experimental import pallas as pl
from jax.experimental.pallas import tpu as pltpu
from jax.experimental.pallas import tpu_sc as plsc

NUM_WORKERS = 32
CHUNKS = (104, 96)
EPAD = 128
TBLK = 32768


QUART = TBLK // 4


def _tpad_body(tt_ref, out_ref):
    t = jnp.transpose(tt_ref[...], (1, 0))
    lo = jax.lax.bitcast_convert_type(t[:, 0:32], jnp.uint32)
    hi = jax.lax.bitcast_convert_type(t[:, 32:64], jnp.uint32)
    w = (hi & jnp.uint32(0xFFFF0000)) | (lo >> 16)
    f = jax.lax.bitcast_convert_type(w, jnp.float32)
    out_ref[:, 0:32] = f[0:QUART]
    out_ref[:, 32:64] = f[QUART:2 * QUART]
    out_ref[:, 64:96] = f[2 * QUART:3 * QUART]
    out_ref[:, 96:128] = f[3 * QUART:TBLK]


def _make_tpad(vocab, embed):
    nblk = pl.cdiv(vocab, TBLK)
    return pl.pallas_call(
        _tpad_body,
        grid=(nblk,),
        in_specs=[pl.BlockSpec((embed, TBLK), lambda i: (0, i))],
        out_specs=pl.BlockSpec((QUART, 128), lambda i: (i, 0)),
        out_shape=jax.ShapeDtypeStruct((nblk * QUART, 128), jnp.float32),
        compiler_params=pltpu.CompilerParams(
            dimension_semantics=("arbitrary",)),
    )


def _make_sc_pool(batch, seq, vocab, embed):
    assert batch % NUM_WORKERS == 0
    b_per_w = batch // NUM_WORKERS
    assert sum(CHUNKS) == seq and all(c % 8 == 0 and c <= 128 for c in CHUNKS)
    assert seq % 8 == 0
    idx_per_w = b_per_w * seq
    assert embed % 32 == 0
    nword = embed // 32

    mesh = plsc.VectorSubcoreMesh(core_axis_name="c", subcore_axis_name="s",
                                  num_cores=2, num_subcores=16)

    @functools.partial(
        pl.kernel,
        out_type=jax.ShapeDtypeStruct((batch, embed), jnp.float32),
        mesh=mesh,
        scratch_types=[
            pltpu.VMEM((idx_per_w,), jnp.int32),
            pltpu.VMEM((seq, embed // 2), jnp.float32),
            pltpu.VMEM((seq, embed // 2), jnp.float32),
            pltpu.VMEM((seq, embed // 2), jnp.float32),
            pltpu.VMEM((b_per_w, embed), jnp.float32),
            pltpu.SemaphoreType.DMA,
            pltpu.SemaphoreType.DMA,
            pltpu.SemaphoreType.DMA,
        ],
        compiler_params=pltpu.CompilerParams(use_tc_tiling_on_sc=False,
                                             needs_layout_passes=False),
    )
    def sc_pool(x_hbm, table_hbm, out_hbm, idx_v, buf_a, buf_b, buf_c,
                sout_v, sem_a, sem_b, sem_c):
        wid = lax.axis_index("s") * 2 + lax.axis_index("c")
        base = wid * b_per_w
        bufs = (buf_a, buf_b, buf_c)
        sems = (sem_a, sem_b, sem_c)

        pltpu.sync_copy(x_hbm.at[pl.ds(base * seq, idx_per_w)], idx_v)

        def fire(row, buf, sem):
            ibase = row * seq
            off = 0
            for c in CHUNKS:
                pltpu.async_copy(
                    table_hbm.at[idx_v.at[pl.ds(ibase + off, c)]],
                    buf.at[pl.ds(off, c)],
                    sem,
                )
                off += c

        def drain(buf, sem):
            pltpu.make_async_copy(table_hbm.at[pl.ds(0, seq)], buf, sem).wait()

        def accumulate(row, buf):
            def step(t, accs):
                rbase = t * 8
                out = list(accs)
                for k in range(nword):
                    sl = pl.ds(k * 16, 16)
                    ev, od = [], []
                    for r in range(8):
                        words = buf[rbase + r, sl]
                        a, b = plsc.unpack(plsc.bitcast(words, jnp.bfloat16),
                                           format=plsc.PackFormat.INTERLEAVED)
                        ev.append(a)
                        od.append(b)
                    se = ((ev[0] + ev[1]) + (ev[2] + ev[3])) + \
                         ((ev[4] + ev[5]) + (ev[6] + ev[7]))
                    so = ((od[0] + od[1]) + (od[2] + od[3])) + \
                         ((od[4] + od[5]) + (od[6] + od[7]))
                    out[2 * k] = out[2 * k] + se
                    out[2 * k + 1] = out[2 * k + 1] + so
                return tuple(out)

            zeros = tuple(jnp.zeros((16,), jnp.float32)
                          for _ in range(2 * nword))
            accs = lax.fori_loop(0, seq // 8, step, zeros)
            for j in range(nword):
                sout_v[row, pl.ds(j * 16, 16)] = accs[2 * j]
                sout_v[row, pl.ds((nword + j) * 16, 16)] = accs[2 * j + 1]

        fire(0, bufs[0], sems[0])
        fire(1, bufs[1], sems[1])

        @pl.loop(0, b_per_w, step=3)
        def _row_loop(i):
            for b in range(3):
                row = i + b
                nxt = row + 2
                fslot = (b + 2) % 3

                @pl.when(nxt < b_per_w)
                def _():
                    fire(nxt, bufs[fslot], sems[fslot])

                @pl.when(row < b_per_w)
                def _():
                    drain(bufs[b], sems[b])
                    accumulate(row, bufs[b])

        pltpu.sync_copy(sout_v, out_hbm.at[pl.ds(base, b_per_w)])

    return sc_pool


def _mlp_body(ssum_ref, mask_ref, w1_ref, b1_ref, w2_ref, b2_ref, out_ref):
    denom = jnp.maximum(jnp.sum(mask_ref[...], axis=1, keepdims=True), 1.0)
    s = ssum_ref[...] / denom
    h = jnp.dot(s, w1_ref[...], preferred_element_type=jnp.float32)
    h = jnp.maximum(h + b1_ref[...], 0.0)
    out_ref[...] = jnp.dot(h, w2_ref[...],
                           preferred_element_type=jnp.float32) + b2_ref[...]


def kernel(x, mask, table, W1, b1, W2, b2):
    batch, seq = x.shape
    vocab, embed = table.shape
    hidden = W1.shape[1]
    nout = W2.shape[1]

    x32 = x.astype(jnp.int32).reshape(-1)
    j = x32 & (TBLK - 1)
    p = ((x32 >> 15) << 13) + (j & (QUART - 1))
    x_flat = (p << 2) + (j >> 13)
    table_pack = _make_tpad(vocab, embed)(table.T)
    table_lin = table_pack.reshape(table_pack.shape[0] * 4, embed // 2)
    ssum = _make_sc_pool(batch, seq, vocab, embed)(x_flat, table_lin)

    blk = 512
    grid = (batch // blk,)
    out = pl.pallas_call(
        _mlp_body,
        grid=grid,
        in_specs=[
            pl.BlockSpec((blk, embed), lambda i: (i, 0)),
            pl.BlockSpec((blk, seq), lambda i: (i, 0)),
            pl.BlockSpec((embed, hidden), lambda i: (0, 0)),
            pl.BlockSpec((1, hidden), lambda i: (0, 0)),
            pl.BlockSpec((hidden, nout), lambda i: (0, 0)),
            pl.BlockSpec((1, nout), lambda i: (0, 0)),
        ],
        out_specs=pl.BlockSpec((blk, nout), lambda i: (i, 0)),
        out_shape=jax.ShapeDtypeStruct((batch, nout), jnp.float32),
    )(ssum, mask, W1, b1.reshape(1, -1), W2, b2.reshape(1, -1))
    return out


# speedup vs baseline: 1.4251x; 1.4251x over previous
import functools

import jax
import jax.numpy as jnp
from jax import lax
from jax.experimental import pallas as pl
from jax.experimental.pallas import tpu as pltpu
from jax.experimental.pallas import tpu_sc as plsc

NUM_WORKERS = 32
CHUNKS = (104, 96)
TBLK = 32768
HALF = TBLK // 2


def _tpad_body(tt_ref, out_ref):
    t = jnp.transpose(tt_ref[...], (1, 0))
    out_ref[:, 0:64] = t[0:HALF]
    out_ref[:, 64:128] = t[HALF:TBLK]


def _make_tpad(vocab, embed):
    nblk = pl.cdiv(vocab, TBLK)
    return pl.pallas_call(
        _tpad_body,
        grid=(nblk,),
        in_specs=[pl.BlockSpec((embed, TBLK), lambda i: (0, i))],
        out_specs=pl.BlockSpec((HALF, 128), lambda i: (i, 0)),
        out_shape=jax.ShapeDtypeStruct((nblk * HALF, 128), jnp.float32),
        compiler_params=pltpu.CompilerParams(
            dimension_semantics=("arbitrary",)),
    )


def _make_sc_pool(batch, seq, vocab, embed):
    assert batch % NUM_WORKERS == 0
    b_per_w = batch // NUM_WORKERS
    assert sum(CHUNKS) == seq and all(c % 8 == 0 and c <= 128 for c in CHUNKS)
    assert seq % 8 == 0
    idx_per_w = b_per_w * seq
    assert embed % 16 == 0
    nvec = embed // 16

    mesh = plsc.VectorSubcoreMesh(core_axis_name="c", subcore_axis_name="s",
                                  num_cores=2, num_subcores=16)

    @functools.partial(
        pl.kernel,
        out_type=jax.ShapeDtypeStruct((batch, embed), jnp.float32),
        mesh=mesh,
        scratch_types=[
            pltpu.VMEM((idx_per_w,), jnp.int32),
            pltpu.VMEM((seq, embed), jnp.float32),
            pltpu.VMEM((seq, embed), jnp.float32),
            pltpu.VMEM((seq, embed), jnp.float32),
            pltpu.VMEM((b_per_w, embed), jnp.float32),
            pltpu.SemaphoreType.DMA,
            pltpu.SemaphoreType.DMA,
            pltpu.SemaphoreType.DMA,
        ],
        compiler_params=pltpu.CompilerParams(use_tc_tiling_on_sc=False),
    )
    def sc_pool(x_hbm, table_hbm, out_hbm, idx_v, buf_a, buf_b, buf_c,
                sout_v, sem_a, sem_b, sem_c):
        wid = lax.axis_index("s") * 2 + lax.axis_index("c")
        base = wid * b_per_w
        bufs = (buf_a, buf_b, buf_c)
        sems = (sem_a, sem_b, sem_c)

        pltpu.sync_copy(x_hbm.at[pl.ds(base * seq, idx_per_w)], idx_v)

        def fire(row, buf, sem):
            ibase = row * seq
            off = 0
            for c in CHUNKS:
                pltpu.async_copy(
                    table_hbm.at[idx_v.at[pl.ds(ibase + off, c)]],
                    buf.at[pl.ds(off, c)],
                    sem,
                )
                off += c

        def drain(buf, sem):
            pltpu.make_async_copy(table_hbm.at[pl.ds(0, seq)], buf, sem).wait()

        def accumulate(row, buf):
            def step(t, accs):
                rbase = t * 8
                out = []
                for k in range(nvec):
                    sl = pl.ds(k * 16, 16)
                    l = [buf[rbase + r, sl] for r in range(8)]
                    s = ((l[0] + l[1]) + (l[2] + l[3])) + \
                        ((l[4] + l[5]) + (l[6] + l[7]))
                    out.append(accs[k] + s)
                return tuple(out)

            zeros = tuple(jnp.zeros((16,), jnp.float32) for _ in range(nvec))
            accs = lax.fori_loop(0, seq // 8, step, zeros)
            for k in range(nvec):
                sout_v[row, pl.ds(k * 16, 16)] = accs[k]

        fire(0, bufs[0], sems[0])
        fire(1, bufs[1], sems[1])

        @pl.loop(0, b_per_w, step=3)
        def _row_loop(i):
            for b in range(3):
                row = i + b
                nxt = row + 2
                fslot = (b + 2) % 3

                @pl.when(nxt < b_per_w)
                def _():
                    fire(nxt, bufs[fslot], sems[fslot])

                @pl.when(row < b_per_w)
                def _():
                    drain(bufs[b], sems[b])
                    accumulate(row, bufs[b])

        pltpu.sync_copy(sout_v, out_hbm.at[pl.ds(base, b_per_w)])

    return sc_pool


def _mlp_body(ssum_ref, mask_ref, w1_ref, b1_ref, w2_ref, b2_ref, out_ref):
    denom = jnp.maximum(jnp.sum(mask_ref[...], axis=1, keepdims=True), 1.0)
    s = ssum_ref[...] / denom
    h = jnp.dot(s, w1_ref[...], preferred_element_type=jnp.float32)
    h = jnp.maximum(h + b1_ref[...], 0.0)
    out_ref[...] = jnp.dot(h, w2_ref[...],
                           preferred_element_type=jnp.float32) + b2_ref[...]


def kernel(x, mask, table, W1, b1, W2, b2):
    batch, seq = x.shape
    vocab, embed = table.shape
    hidden = W1.shape[1]
    nout = W2.shape[1]

    x32 = x.astype(jnp.int32).reshape(-1)
    j = x32 & (TBLK - 1)
    p = ((x32 >> 15) << 14) + (j & (HALF - 1))
    x_flat = (p << 1) + (j >> 14)
    table_pack = _make_tpad(vocab, embed)(table.T)
    table_lin = table_pack.reshape(table_pack.shape[0] * 2, embed)
    ssum = _make_sc_pool(batch, seq, vocab, embed)(x_flat, table_lin)

    blk = 512
    grid = (batch // blk,)
    out = pl.pallas_call(
        _mlp_body,
        grid=grid,
        in_specs=[
            pl.BlockSpec((blk, embed), lambda i: (i, 0)),
            pl.BlockSpec((blk, seq), lambda i: (i, 0)),
            pl.BlockSpec((embed, hidden), lambda i: (0, 0)),
            pl.BlockSpec((1, hidden), lambda i: (0, 0)),
            pl.BlockSpec((hidden, nout), lambda i: (0, 0)),
            pl.BlockSpec((1, nout), lambda i: (0, 0)),
        ],
        out_specs=pl.BlockSpec((blk, nout), lambda i: (i, 0)),
        out_shape=jax.ShapeDtypeStruct((batch, nout), jnp.float32),
    )(ssum, mask, W1, b1.reshape(1, -1), W2, b2.reshape(1, -1))
    return out
